# baseline (speedup 1.0000x reference)
import jax
import jax.numpy as jnp
from jax.experimental import pallas as pl
from jax.experimental.pallas import tpu as pltpu

_NUM_EMB = 1024
_DIM = 256
_NB = 8
_NP = 128
_BLK = _NB * _NP
_COMMIT = 0.25


def _vq_tc_kernel(x_ref, e_ref, enc_ref, q_ref, loss_ref):
    i = pl.program_id(0)
    x = x_ref[...].reshape(_BLK, _DIM)
    e = e_ref[...]
    term1 = jnp.sum(x * x, axis=1, keepdims=True)
    term2 = jnp.sum(e * e, axis=1)
    term3 = jnp.dot(x, e.T, preferred_element_type=jnp.float32)
    dist = (term1 + term2[None, :]) - 2.0 * term3
    min_d = jnp.min(dist, axis=1, keepdims=True)
    col = jax.lax.broadcasted_iota(jnp.int32, (_BLK, _NUM_EMB), 1)
    idx = jnp.min(jnp.where(dist == min_d, col, _NUM_EMB), axis=1)
    enc = (col == idx[:, None]).astype(jnp.float32)
    enc_ref[...] = enc.reshape(_NB, _NP, _NUM_EMB)
    q = jnp.dot(enc, e, preferred_element_type=jnp.float32)
    q_ref[...] = q.reshape(_NB, _NP * _DIM)
    part = jnp.sum(min_d)

    @pl.when(i == 0)
    def _():
        loss_ref[0, 0] = 0.0

    loss_ref[0, 0] += part


def kernel(inputs, embedding):
    b, t, d = inputs.shape
    n = b * t
    n_pc = t // _NP
    n_bg = b // _NB
    grid = (n_bg * n_pc,)
    enc3, q, loss_sum = pl.pallas_call(
        _vq_tc_kernel,
        grid=grid,
        in_specs=[
            pl.BlockSpec((_NB, _NP, _DIM), lambda i, n_pc=n_pc: (i // n_pc, i % n_pc, 0)),
            pl.BlockSpec((_NUM_EMB, _DIM), lambda i: (0, 0)),
        ],
        out_specs=[
            pl.BlockSpec((_NB, _NP, _NUM_EMB), lambda i, n_pc=n_pc: (i // n_pc, i % n_pc, 0)),
            pl.BlockSpec((_NB, _NP * _DIM), lambda i, n_pc=n_pc: (i // n_pc, i % n_pc)),
            pl.BlockSpec((1, 1), lambda i: (0, 0), memory_space=pltpu.SMEM),
        ],
        out_shape=[
            jax.ShapeDtypeStruct((b, t, _NUM_EMB), jnp.float32),
            jax.ShapeDtypeStruct((b, t * _DIM), jnp.float32),
            jax.ShapeDtypeStruct((1, 1), jnp.float32),
        ],
    )(inputs, embedding)
    loss = loss_sum[0, 0] * ((1.0 + _COMMIT) / (n * _DIM))
    enc = enc3.reshape(n, _NUM_EMB)
    return (loss, q, enc)


# speedup vs baseline: 1.0479x; 1.0479x over previous
import jax
import jax.numpy as jnp
from jax.experimental import pallas as pl
from jax.experimental.pallas import tpu as pltpu

_NUM_EMB = 1024
_DIM = 256
_NB = 8
_NP = 256
_BLK = _NB * _NP
_COMMIT = 0.25


def _vq_tc_kernel(x_ref, e_ref, enc_ref, q_ref, loss_ref):
    i = pl.program_id(0)
    x = x_ref[...].reshape(_BLK, _DIM)
    e = e_ref[...]
    term1 = jnp.sum(x * x, axis=1, keepdims=True)
    term2 = jnp.sum(e * e, axis=1)
    term3 = jnp.dot(x, e.T, preferred_element_type=jnp.float32)
    dist = (term1 + term2[None, :]) - 2.0 * term3
    min_d = jnp.min(dist, axis=1, keepdims=True)
    col = jax.lax.broadcasted_iota(jnp.int32, (_BLK, _NUM_EMB), 1)
    idx = jnp.min(jnp.where(dist == min_d, col, _NUM_EMB), axis=1)
    enc = (col == idx[:, None]).astype(jnp.float32)
    enc_ref[...] = enc.reshape(_NB, _NP, _NUM_EMB)
    q = jnp.dot(enc, e, preferred_element_type=jnp.float32)
    q_ref[...] = q.reshape(_NB, _NP * _DIM)
    part = jnp.sum(min_d)

    @pl.when(i == 0)
    def _():
        loss_ref[0, 0] = 0.0

    loss_ref[0, 0] += part


def kernel(inputs, embedding):
    b, t, d = inputs.shape
    n = b * t
    n_pc = t // _NP
    n_bg = b // _NB
    grid = (n_bg * n_pc,)
    enc3, q, loss_sum = pl.pallas_call(
        _vq_tc_kernel,
        grid=grid,
        in_specs=[
            pl.BlockSpec((_NB, _NP, _DIM), lambda i, n_pc=n_pc: (i // n_pc, i % n_pc, 0)),
            pl.BlockSpec((_NUM_EMB, _DIM), lambda i: (0, 0)),
        ],
        out_specs=[
            pl.BlockSpec((_NB, _NP, _NUM_EMB), lambda i, n_pc=n_pc: (i // n_pc, i % n_pc, 0)),
            pl.BlockSpec((_NB, _NP * _DIM), lambda i, n_pc=n_pc: (i // n_pc, i % n_pc)),
            pl.BlockSpec((1, 1), lambda i: (0, 0), memory_space=pltpu.SMEM),
        ],
        out_shape=[
            jax.ShapeDtypeStruct((b, t, _NUM_EMB), jnp.float32),
            jax.ShapeDtypeStruct((b, t * _DIM), jnp.float32),
            jax.ShapeDtypeStruct((1, 1), jnp.float32),
        ],
    )(inputs, embedding)
    loss = loss_sum[0, 0] * ((1.0 + _COMMIT) / (n * _DIM))
    enc = enc3.reshape(n, _NUM_EMB)
    return (loss, q, enc)
